# Initial kernel scaffold; baseline (speedup 1.0000x reference)
import jax
import jax.numpy as jnp
from jax.experimental import pallas as pl


def kernel(input_id, table):
    raise NotImplementedError("write your pallas kernel here")


# speedup vs baseline: 1.7944x; 1.7944x over previous
import functools

import jax
import jax.numpy as jnp
from jax import lax
from jax.experimental import pallas as pl
from jax.experimental.pallas import tpu as pltpu
from jax.experimental.pallas import tpu_sc as plsc

NC = 2
NS = 16
NW = NC * NS

EMBED_DIM = 64
CHUNK = 512


@functools.partial(jax.jit, static_argnames=("total",))
def _gather_rows(idx_flat, table, total):
    b_per_w = total // NW
    n_chunks = b_per_w // CHUNK
    mesh = plsc.VectorSubcoreMesh(core_axis_name="c", subcore_axis_name="s")

    @functools.partial(
        pl.kernel,
        out_type=jax.ShapeDtypeStruct((total, EMBED_DIM), jnp.float32),
        mesh=mesh,
        scratch_types=[
            pltpu.VMEM((CHUNK,), jnp.int32),
            pltpu.VMEM((CHUNK, EMBED_DIM), jnp.float32),
            pltpu.SemaphoreType.DMA,
        ],
        compiler_params=pltpu.CompilerParams(use_tc_tiling_on_sc=False),
    )
    def body(idx_hbm, table_hbm, out_hbm, idx_v, rows_v, sem):
        wid = lax.axis_index("s") * NC + lax.axis_index("c")
        base = wid * b_per_w

        def chunk_body(i, carry):
            off = base + i * CHUNK
            pltpu.sync_copy(idx_hbm.at[pl.ds(off, CHUNK)], idx_v)
            pltpu.async_copy(table_hbm.at[idx_v], rows_v, sem).wait()
            pltpu.sync_copy(rows_v, out_hbm.at[pl.ds(off, CHUNK)])
            return carry

        lax.fori_loop(0, n_chunks, chunk_body, 0)

    return body(idx_flat, table)


def kernel(input_id, table):
    batch, hist = input_id.shape
    total = batch * hist
    idx_flat = input_id.reshape(total).astype(jnp.int32)
    out = _gather_rows(idx_flat, table.astype(jnp.float32), total)
    return out.reshape(batch, hist, EMBED_DIM)


# speedup vs baseline: 1.8712x; 1.0428x over previous
import functools

import jax
import jax.numpy as jnp
from jax import lax
from jax.experimental import pallas as pl
from jax.experimental.pallas import tpu as pltpu
from jax.experimental.pallas import tpu_sc as plsc

NC = 2
NS = 16
NW = NC * NS

EMBED_DIM = 64
CHUNK = 800


@functools.partial(jax.jit, static_argnames=("total",))
def _gather_rows(idx_flat, table, total):
    b_per_w = total // NW
    n_chunks = b_per_w // CHUNK
    mesh = plsc.VectorSubcoreMesh(core_axis_name="c", subcore_axis_name="s")

    @functools.partial(
        pl.kernel,
        out_type=jax.ShapeDtypeStruct((total, EMBED_DIM), jnp.float32),
        mesh=mesh,
        scratch_types=[
            pltpu.VMEM((b_per_w,), jnp.int32),
            pltpu.VMEM((CHUNK, EMBED_DIM), jnp.float32),
            pltpu.VMEM((CHUNK, EMBED_DIM), jnp.float32),
            pltpu.SemaphoreType.DMA,
            pltpu.SemaphoreType.DMA,
            pltpu.SemaphoreType.DMA,
            pltpu.SemaphoreType.DMA,
        ],
        compiler_params=pltpu.CompilerParams(use_tc_tiling_on_sc=False),
    )
    def body(idx_hbm, table_hbm, out_hbm, idx_v, r0, r1, sg0, sg1, ss0, ss1):
        rows = [r0, r1]
        sg = [sg0, sg1]
        ss = [ss0, ss1]
        wid = lax.axis_index("s") * NC + lax.axis_index("c")
        base = wid * b_per_w
        pltpu.sync_copy(idx_hbm.at[pl.ds(base, b_per_w)], idx_v)

        def g_copy(i, b):
            return pltpu.make_async_copy(
                table_hbm.at[idx_v.at[pl.ds(i * CHUNK, CHUNK)]], rows[b], sg[b])

        def s_copy(i, b):
            return pltpu.make_async_copy(
                rows[b], out_hbm.at[pl.ds(base + i * CHUNK, CHUNK)], ss[b])

        g_copy(0, 0).start()

        def outer(p, carry):
            for b in range(2):
                i = 2 * p + b

                @pl.when(i + 1 < n_chunks)
                def _prefetch():
                    @pl.when(i >= 1)
                    def _free():
                        s_copy(i - 1, 1 - b).wait()
                    g_copy(i + 1, 1 - b).start()

                g_copy(i, b).wait()
                s_copy(i, b).start()
            return carry

        lax.fori_loop(0, n_chunks // 2, outer, 0)
        s_copy(n_chunks - 2, 0).wait()
        s_copy(n_chunks - 1, 1).wait()

    return body(idx_flat, table)


def kernel(input_id, table):
    batch, hist = input_id.shape
    total = batch * hist
    idx_flat = input_id.reshape(total).astype(jnp.int32)
    out = _gather_rows(idx_flat, table.astype(jnp.float32), total)
    return out.reshape(batch, hist, EMBED_DIM)
